# baseline (speedup 1.0000x reference)
import functools

import jax
import jax.numpy as jnp
from jax import lax
from jax.experimental import pallas as pl
from jax.experimental.pallas import tpu as pltpu
from jax.experimental.pallas import tpu_sc as plsc

_N = 10000
_E = 320000
_D = 128
_G = 128
_K = 512
_C = 10
_GAMMA = 0.5
_CW = 0.25

_CHUNK = 128
_NC, _NS = 2, 16
_NW = _NC * _NS
_NCHP = 2560
_EP = _NCHP * _CHUNK
_NT1 = _NCHP // _NW
_NT2 = _NCHP // _NS
_W = 40

_ZR = 632
_ACCR = _ZR * _NS
_WR = 624

_BN = 2000
_NB = _N // _BN

_HI = jax.lax.Precision.HIGHEST

_MESH = dict(core_axis_name="c", subcore_axis_name="s",
             num_cores=_NC, num_subcores=_NS)

_SEG_SCRATCH = [
    pltpu.VMEM((_W, _CHUNK), jnp.int32),
    pltpu.VMEM((_W, _CHUNK), jnp.int32),
    pltpu.VMEM((_CHUNK, _D), jnp.float32),
    pltpu.VMEM((_CHUNK, _D), jnp.float32),
    pltpu.VMEM_SHARED((_ACCR, _D), jnp.float32),
    pltpu.SemaphoreType.DMA,
    pltpu.SemaphoreType.DMA,
    pltpu.SemaphoreType.DMA,
    pltpu.SemaphoreType.DMA,
]


def _edge_loop(h_hbm, sidx, didx, rows0, rows1, sem0, sem1, ssem0, ssem1,
               acc, n):
    pltpu.async_copy(h_hbm.at[sidx.at[0]], rows0, sem0)
    pltpu.async_copy(h_hbm.at[sidx.at[1]], rows1, sem1)

    def body(jj, carry):
        j0 = jj * 2
        pltpu.make_async_copy(h_hbm.at[sidx.at[j0]], rows0, sem0).wait()
        pltpu.async_copy(rows0, acc.at[didx.at[j0]], ssem0, add=True)
        pltpu.make_async_copy(h_hbm.at[sidx.at[j0 + 1]], rows1, sem1).wait()
        pltpu.async_copy(rows1, acc.at[didx.at[j0 + 1]], ssem1, add=True)
        pltpu.make_async_copy(rows0, acc.at[didx.at[j0]], ssem0).wait()

        @pl.when(j0 + 2 < n)
        def _():
            pltpu.async_copy(h_hbm.at[sidx.at[j0 + 2]], rows0, sem0)

        pltpu.make_async_copy(rows1, acc.at[didx.at[j0 + 1]], ssem1).wait()

        @pl.when(j0 + 3 < n)
        def _():
            pltpu.async_copy(h_hbm.at[sidx.at[j0 + 3]], rows1, sem1)

        return carry

    lax.fori_loop(0, n // 2, body, 0)


def _zero_acc(rows0, acc, s):
    def zrow(r, carry):
        for l in range(8):
            rows0[r, pl.ds(l * 16, 16)] = jnp.zeros((16,), jnp.float32)
        return carry

    lax.fori_loop(0, _CHUNK, zrow, 0)
    for t in range(4):
        off = pl.multiple_of(s * _ZR + _CHUNK * t, 8)
        pltpu.sync_copy(rows0, acc.at[pl.ds(off, _CHUNK)])
    off = pl.multiple_of(s * _ZR + 4 * _CHUNK, 8)
    pltpu.sync_copy(rows0.at[pl.ds(0, _ZR - 4 * _CHUNK)],
                    acc.at[pl.ds(off, _ZR - 4 * _CHUNK)])


def _window_loop(h_hbm, fetch_idx, base, nwin,
                 sidx, didx, rows0, rows1, sem0, sem1, ssem0, ssem1, acc):
    def wbody(wi, carry):
        off = pl.multiple_of(base + wi * _W, 8)
        fetch_idx(off, sidx, didx)
        _edge_loop(h_hbm, sidx, didx, rows0, rows1, sem0, sem1,
                   ssem0, ssem1, acc, _W)
        return carry

    lax.fori_loop(0, nwin, wbody, 0)


def _acc_writeout(acc, out_hbm, c, s):
    off = pl.multiple_of(s * _WR, 8)

    @pl.when(s < _NS - 1)
    def _():
        pltpu.sync_copy(acc.at[pl.ds(off, _WR)],
                        out_hbm.at[c, pl.ds(off, _WR)])

    @pl.when(s == _NS - 1)
    def _():
        last = (_NS - 1) * _WR
        pltpu.sync_copy(acc.at[pl.ds(last, _N - last)],
                        out_hbm.at[c, pl.ds(last, _N - last)])


def _segsum_sc(h, src2d, dst2d):
    @functools.partial(
        pl.kernel,
        out_type=jax.ShapeDtypeStruct((_NC, _N, _D), jnp.float32),
        mesh=plsc.VectorSubcoreMesh(**_MESH),
        scratch_types=_SEG_SCRATCH,
    )
    def k(h_hbm, src_hbm, dst_hbm, out_hbm,
          sidx, didx, rows0, rows1, acc, sem0, sem1, ssem0, ssem1):
        c = lax.axis_index("c")
        s = lax.axis_index("s")
        _zero_acc(rows0, acc, s)
        plsc.subcore_barrier()

        def fetch(off, si, di):
            pltpu.sync_copy(src_hbm.at[pl.ds(off, _W)], si)
            pltpu.sync_copy(dst_hbm.at[pl.ds(off, _W)], di)

        _window_loop(h_hbm, fetch, (c * _NS + s) * _NT1, _NT1 // _W,
                     sidx, didx, rows0, rows1, sem0, sem1, ssem0, ssem1, acc)
        plsc.subcore_barrier()
        _acc_writeout(acc, out_hbm, c, s)

    return k(h, src2d, dst2d)


def _segsum_dual_sc(h2, srcab, dst2d):
    @functools.partial(
        pl.kernel,
        out_type=jax.ShapeDtypeStruct((_NC, _N, _D), jnp.float32),
        mesh=plsc.VectorSubcoreMesh(**_MESH),
        scratch_types=_SEG_SCRATCH,
    )
    def k(h_hbm, src_hbm, dst_hbm, out_hbm,
          sidx, didx, rows0, rows1, acc, sem0, sem1, ssem0, ssem1):
        c = lax.axis_index("c")
        s = lax.axis_index("s")
        _zero_acc(rows0, acc, s)
        plsc.subcore_barrier()

        def fetch(off, si, di):
            pltpu.sync_copy(src_hbm.at[c, pl.ds(off, _W)], si)
            pltpu.sync_copy(dst_hbm.at[pl.ds(off, _W)], di)

        _window_loop(h_hbm, fetch, s * _NT2, _NT2 // _W,
                     sidx, didx, rows0, rows1, sem0, sem1, ssem0, ssem1, acc)
        plsc.subcore_barrier()
        _acc_writeout(acc, out_hbm, c, s)

    return k(h2, srcab, dst2d)


def _gin_layer2_tc(h_and_spec, aggs, eps2, w1s, b1s, w2s, b2s,
                   premlp=None):
    n_agg = len(aggs[1])

    def body(*refs):
        eps_ref = refs[0]
        h_ref = refs[1]
        agg_refs = refs[2:2 + n_agg]
        w1_ref, b1_ref, w2_ref, b2_ref = refs[2 + n_agg:6 + n_agg]
        if premlp is None:
            (o_ref,) = refs[6 + n_agg:]
        else:
            pw_ref, pb_ref, o_ref, hp_ref, st_ref = refs[6 + n_agg:]
        agg = agg_refs[0][0]
        for r in agg_refs[1:]:
            agg = agg + r[0]
        z = (1.0 + eps_ref[0, 0]) * h_ref[...] + agg
        t = jnp.dot(z, w1_ref[0], preferred_element_type=jnp.float32)
        t = jnp.maximum(t + b1_ref[0], 0.0)
        o = jnp.dot(t, w2_ref[0], preferred_element_type=jnp.float32)
        o = jnp.maximum(o + b2_ref[0], 0.0)
        o_ref[...] = o

        if premlp is not None:
            j = pl.program_id(0)
            i = pl.program_id(1)

            @pl.when(jnp.logical_and(j == 0, i == 0))
            def _():
                st_ref[...] = jnp.zeros_like(st_ref)

            @pl.when(j == 0)
            def _():
                hp = jnp.dot(o, pw_ref[...],
                             preferred_element_type=jnp.float32) + pb_ref[...]
                hp_ref[...] = hp
                st_ref[0:1, :] += jnp.sum(hp, axis=0, keepdims=True)
                st_ref[1:2, :] += jnp.sum(hp * hp, axis=0, keepdims=True)

    h2, h_spec = h_and_spec
    agg_arr, agg_specs = aggs
    in_specs = [
        pl.BlockSpec(memory_space=pltpu.SMEM),
        h_spec,
        *agg_specs,
        pl.BlockSpec((1, _D, 2 * _D), lambda j, i: (j, 0, 0)),
        pl.BlockSpec((1, 1, 2 * _D), lambda j, i: (j, 0, 0)),
        pl.BlockSpec((1, 2 * _D, _D), lambda j, i: (j, 0, 0)),
        pl.BlockSpec((1, 1, _D), lambda j, i: (j, 0, 0)),
    ]
    out_specs = [pl.BlockSpec((_BN, _D), lambda j, i: (j * _NB + i, 0))]
    out_shape = [jax.ShapeDtypeStruct((2 * _N, _D), jnp.float32)]
    args = [eps2, h2, *agg_arr, w1s, b1s.reshape(2, 1, -1), w2s,
            b2s.reshape(2, 1, -1)]
    if premlp is not None:
        pw, pb = premlp
        in_specs += [
            pl.BlockSpec((_D, 2 * _D), lambda j, i: (0, 0)),
            pl.BlockSpec((1, 2 * _D), lambda j, i: (0, 0)),
        ]
        out_specs += [
            pl.BlockSpec((_BN, 2 * _D),
                         lambda j, i: (jnp.where(j == 0, i, _NB), 0)),
            pl.BlockSpec((8, 2 * _D), lambda j, i: (0, 0)),
        ]
        out_shape += [
            jax.ShapeDtypeStruct((_N + _BN, 2 * _D), jnp.float32),
            jax.ShapeDtypeStruct((8, 2 * _D), jnp.float32),
        ]
        args += [pw, pb.reshape(1, -1)]
    out = pl.pallas_call(
        body,
        grid=(2, _NB),
        in_specs=in_specs,
        out_specs=out_specs,
        out_shape=out_shape,
    )(*args)
    return out[0] if premlp is None else out


def _epilogue_tc(hp, stats, bn_g, bn_b, w2, b2, nf2, cb, batch2d,
                 cls_w, cls_b):
    def body(hp_ref, st_ref, g_ref, b_ref, w2_ref, b2_ref, nf_ref, cb_ref,
             bat_ref, w_ref, cb2_ref, cg_ref, sg_ref, sm_ref, cm_ref,
             logit_ref, cgo_ref, sgo_ref, cmt_ref, reg_ref):
        i = pl.program_id(0)

        @pl.when(i == 0)
        def _():
            cg_ref[...] = jnp.zeros_like(cg_ref)
            sg_ref[...] = jnp.zeros_like(sg_ref)
            sm_ref[...] = jnp.zeros_like(sm_ref)
            cm_ref[...] = jnp.zeros_like(cm_ref)

        m = st_ref[0:1, :] * (1.0 / _N)
        v = st_ref[1:2, :] * (1.0 / _N) - m * m
        hpn = (hp_ref[...] - m) * lax.rsqrt(v + 1e-5) * g_ref[...] + b_ref[...]
        hpn = jnp.maximum(hpn, 0.0)
        logits = jnp.dot(hpn, w2_ref[...],
                         preferred_element_type=jnp.float32) + b2_ref[...]
        score = jax.nn.sigmoid(logits)
        pos = jnp.mean(score, axis=1, keepdims=True)

        nfb = nf_ref[...]
        cbm = cb_ref[...]
        cbsq = lax.dot_general(jnp.ones((1, _D), jnp.float32), cbm * cbm,
                               (((1,), (1,)), ((), ())), precision=_HI,
                               preferred_element_type=jnp.float32)
        prox = lax.dot_general(nfb, cbm, (((1,), (1,)), ((), ())),
                               preferred_element_type=jnp.float32)
        dd = cbsq - 2.0 * prox
        minv = jnp.min(dd, axis=1, keepdims=True)
        ik = lax.broadcasted_iota(jnp.int32, dd.shape, 1)
        idx = jnp.min(jnp.where(dd == minv, ik, _K), axis=1, keepdims=True)
        onehot = (ik == idx).astype(jnp.float32)
        q = jnp.dot(onehot, cbm,
                    preferred_element_type=jnp.float32)

        diff = nfb - q
        res = nfb + q
        cpart = res * score
        spart = res * (1.0 - score)

        oh = (lax.broadcasted_iota(jnp.int32, (hp_ref.shape[0], _G), 1)
              == bat_ref[...]).astype(jnp.float32)
        cg_ref[...] += lax.dot_general(oh, cpart, (((0,), (0,)), ((), ())),
                                       preferred_element_type=jnp.float32)
        sg_ref[...] += lax.dot_general(oh, spart, (((0,), (0,)), ((), ())),
                                       preferred_element_type=jnp.float32)
        ph = pos.astype(jnp.bfloat16).astype(jnp.float32)
        rhs = jnp.concatenate(
            [ph, pos - ph, jnp.ones_like(pos),
             jnp.zeros((pos.shape[0], 5), jnp.float32)], axis=1)
        sm_ref[...] += lax.dot_general(oh, rhs, (((0,), (0,)), ((), ())),
                                       preferred_element_type=jnp.float32)
        cm_ref[...] += jnp.full((1, _D), jnp.sum(diff * diff))

        @pl.when(i == _NB - 1)
        def _():
            pos_sum = sm_ref[:, 0:1] + sm_ref[:, 1:2]
            pos_b = pos_sum + 1e-8
            neg_b = sm_ref[:, 2:3] - pos_sum + 1e-8
            inv = 1.0 / jnp.maximum(sm_ref[:, 2:3], 1.0)
            cg = cg_ref[...] * inv
            sg = sg_ref[...] * inv
            cgo_ref[...] = cg
            sgo_ref[...] = sg
            logit_ref[...] = jnp.dot(
                cg, w_ref[...], preferred_element_type=jnp.float32) + cb2_ref[...]
            reg = jnp.mean(jnp.abs(pos_b / (pos_b + neg_b) - _GAMMA))
            reg_ref[...] = jnp.full((1, 1), reg)
            cmt_ref[...] = cm_ref[0:1, 0:1] * (_CW / (_N * _D))

    _c0 = lambda i: (0, 0)
    return pl.pallas_call(
        body,
        grid=(_NB,),
        in_specs=[
            pl.BlockSpec((_BN, 2 * _D), lambda i: (i, 0)),
            pl.BlockSpec((8, 2 * _D), _c0),
            pl.BlockSpec((1, 2 * _D), _c0),
            pl.BlockSpec((1, 2 * _D), _c0),
            pl.BlockSpec((2 * _D, _D), _c0),
            pl.BlockSpec((1, _D), _c0),
            pl.BlockSpec((_BN, _D), lambda i: (_NB + i, 0)),
            pl.BlockSpec((_K, _D), _c0),
            pl.BlockSpec((_BN, 1), lambda i: (i, 0)),
            pl.BlockSpec((_D, _C), _c0),
            pl.BlockSpec((1, _C), _c0),
        ],
        out_specs=[
            pl.BlockSpec((_G, _D), _c0),
            pl.BlockSpec((_G, _D), _c0),
            pl.BlockSpec((_G, 8), _c0),
            pl.BlockSpec((1, _D), _c0),
            pl.BlockSpec((_G, _C), _c0),
            pl.BlockSpec((_G, _D), _c0),
            pl.BlockSpec((_G, _D), _c0),
            pl.BlockSpec((1, 1), _c0),
            pl.BlockSpec((1, 1), _c0),
        ],
        out_shape=[
            jax.ShapeDtypeStruct((_G, _D), jnp.float32),
            jax.ShapeDtypeStruct((_G, _D), jnp.float32),
            jax.ShapeDtypeStruct((_G, 8), jnp.float32),
            jax.ShapeDtypeStruct((1, _D), jnp.float32),
            jax.ShapeDtypeStruct((_G, _C), jnp.float32),
            jax.ShapeDtypeStruct((_G, _D), jnp.float32),
            jax.ShapeDtypeStruct((_G, _D), jnp.float32),
            jax.ShapeDtypeStruct((1, 1), jnp.float32),
            jax.ShapeDtypeStruct((1, 1), jnp.float32),
        ],
    )(hp, stats, bn_g.reshape(1, -1), bn_b.reshape(1, -1), w2,
      b2.reshape(1, -1), nf2, cb, batch2d, cls_w, cls_b.reshape(1, -1))


def _stack2(pa, pb, name):
    return jnp.stack([pa[name], pb[name]])


def kernel(x, edge_index, batch, params):
    npad = _EP - _E
    pad_src = jnp.arange(npad, dtype=jnp.int32) % _N
    pad_dst = _N + jnp.arange(npad, dtype=jnp.int32) % (_ACCR - _N)
    src2d = jnp.concatenate(
        [edge_index[0].astype(jnp.int32), pad_src]).reshape(_NCHP, _CHUNK)
    dst2d = jnp.concatenate(
        [edge_index[1].astype(jnp.int32), pad_dst]).reshape(_NCHP, _CHUNK)
    srcab = jnp.stack([src2d, src2d + _N])
    batch2d = batch.reshape(_N, 1).astype(jnp.int32)

    sep1, sep2 = params["sep_gnn"]
    enc1, enc2 = params["enc_gnn"]
    eps1 = jnp.stack([sep1["eps"], enc1["eps"]]).reshape(2, 1).astype(jnp.float32)
    eps2 = jnp.stack([sep2["eps"], enc2["eps"]]).reshape(2, 1).astype(jnp.float32)

    aggx = _segsum_sc(x, src2d, dst2d)
    h2 = _gin_layer2_tc(
        (x, pl.BlockSpec((_BN, _D), lambda j, i: (i, 0))),
        ([aggx, aggx],
         [pl.BlockSpec((1, _BN, _D), lambda j, i: (0, i, 0)),
          pl.BlockSpec((1, _BN, _D), lambda j, i: (1, i, 0))]),
        eps1, _stack2(sep1, enc1, "w1"), _stack2(sep1, enc1, "b1"),
        _stack2(sep1, enc1, "w2"), _stack2(sep1, enc1, "b2"))
    agg2 = _segsum_dual_sc(h2, srcab, dst2d)
    nf2, hp, stats = _gin_layer2_tc(
        (h2, pl.BlockSpec((_BN, _D), lambda j, i: (j * _NB + i, 0))),
        ([agg2], [pl.BlockSpec((1, _BN, _D), lambda j, i: (j, i, 0))]),
        eps2, _stack2(sep2, enc2, "w1"), _stack2(sep2, enc2, "b1"),
        _stack2(sep2, enc2, "w2"), _stack2(sep2, enc2, "b2"),
        premlp=(params["sep_w1"], params["sep_b1"]))

    outs = _epilogue_tc(
        hp, stats, params["sep_bn_g"], params["sep_bn_b"], params["sep_w2"],
        params["sep_b2"], nf2, params["codebook"], batch2d,
        params["cls_w"], params["cls_b"])
    logit, c_g, s_g, cmt, reg = outs[4:]
    return (logit, c_g, s_g, cmt[0, 0], reg[0, 0])


# speedup vs baseline: 1.2703x; 1.2703x over previous
import functools

import jax
import jax.numpy as jnp
from jax import lax
from jax.experimental import pallas as pl
from jax.experimental.pallas import tpu as pltpu
from jax.experimental.pallas import tpu_sc as plsc

_N = 10000
_E = 320000
_D = 128
_G = 128
_K = 512
_C = 10
_GAMMA = 0.5
_CW = 0.25

_CHUNK = 128
_NC, _NS = 2, 16
_NW = _NC * _NS
_NCHP = 2560
_EP = _NCHP * _CHUNK
_NT1 = _NCHP // _NW
_NT2 = _NCHP // _NS
_W = 40

_ZR = 632
_ACCR = _ZR * _NS
_WR = 624

_BN = 2000
_NB = _N // _BN

_HI = jax.lax.Precision.HIGHEST

_MESH = dict(core_axis_name="c", subcore_axis_name="s",
             num_cores=_NC, num_subcores=_NS)

_SEG_SCRATCH = [
    pltpu.VMEM((_W, _CHUNK), jnp.int32),
    pltpu.VMEM((_W, _CHUNK), jnp.int32),
    pltpu.VMEM((_CHUNK, _D), jnp.float32),
    pltpu.VMEM((_CHUNK, _D), jnp.float32),
    pltpu.VMEM_SHARED((_ACCR, _D), jnp.float32),
    pltpu.SemaphoreType.DMA,
    pltpu.SemaphoreType.DMA,
]


def _edge_loop(h_hbm, sidx, didx, rows0, rows1, sem0, sem1, acc, n):
    pltpu.async_copy(h_hbm.at[sidx.at[0]], rows0, sem0)

    def body(jj, carry):
        j0 = jj * 2
        pltpu.async_copy(h_hbm.at[sidx.at[j0 + 1]], rows1, sem1)
        pltpu.make_async_copy(h_hbm.at[sidx.at[j0]], rows0, sem0).wait()
        pltpu.sync_copy(rows0, acc.at[didx.at[j0]], add=True)

        @pl.when(j0 + 2 < n)
        def _():
            pltpu.async_copy(h_hbm.at[sidx.at[j0 + 2]], rows0, sem0)

        pltpu.make_async_copy(h_hbm.at[sidx.at[j0 + 1]], rows1, sem1).wait()
        pltpu.sync_copy(rows1, acc.at[didx.at[j0 + 1]], add=True)
        return carry

    lax.fori_loop(0, n // 2, body, 0)


def _zero_acc(rows0, acc, s):
    def zrow(r, carry):
        for l in range(8):
            rows0[r, pl.ds(l * 16, 16)] = jnp.zeros((16,), jnp.float32)
        return carry

    lax.fori_loop(0, _CHUNK, zrow, 0)
    for t in range(4):
        off = pl.multiple_of(s * _ZR + _CHUNK * t, 8)
        pltpu.sync_copy(rows0, acc.at[pl.ds(off, _CHUNK)])
    off = pl.multiple_of(s * _ZR + 4 * _CHUNK, 8)
    pltpu.sync_copy(rows0.at[pl.ds(0, _ZR - 4 * _CHUNK)],
                    acc.at[pl.ds(off, _ZR - 4 * _CHUNK)])


def _window_loop(h_hbm, fetch_idx, base, nwin,
                 sidx, didx, rows0, rows1, sem0, sem1, acc):
    def wbody(wi, carry):
        off = pl.multiple_of(base + wi * _W, 8)
        fetch_idx(off, sidx, didx)
        _edge_loop(h_hbm, sidx, didx, rows0, rows1, sem0, sem1, acc, _W)
        return carry

    lax.fori_loop(0, nwin, wbody, 0)


def _acc_writeout(acc, out_hbm, c, s):
    off = pl.multiple_of(s * _WR, 8)

    @pl.when(s < _NS - 1)
    def _():
        pltpu.sync_copy(acc.at[pl.ds(off, _WR)],
                        out_hbm.at[c, pl.ds(off, _WR)])

    @pl.when(s == _NS - 1)
    def _():
        last = (_NS - 1) * _WR
        pltpu.sync_copy(acc.at[pl.ds(last, _N - last)],
                        out_hbm.at[c, pl.ds(last, _N - last)])


def _segsum_sc(h, src2d, dst2d):
    @functools.partial(
        pl.kernel,
        out_type=jax.ShapeDtypeStruct((_NC, _N, _D), jnp.float32),
        mesh=plsc.VectorSubcoreMesh(**_MESH),
        scratch_types=_SEG_SCRATCH,
    )
    def k(h_hbm, src_hbm, dst_hbm, out_hbm,
          sidx, didx, rows0, rows1, acc, sem0, sem1):
        c = lax.axis_index("c")
        s = lax.axis_index("s")
        _zero_acc(rows0, acc, s)
        plsc.subcore_barrier()

        def fetch(off, si, di):
            pltpu.sync_copy(src_hbm.at[pl.ds(off, _W)], si)
            pltpu.sync_copy(dst_hbm.at[pl.ds(off, _W)], di)

        _window_loop(h_hbm, fetch, (c * _NS + s) * _NT1, _NT1 // _W,
                     sidx, didx, rows0, rows1, sem0, sem1, acc)
        plsc.subcore_barrier()
        _acc_writeout(acc, out_hbm, c, s)

    return k(h, src2d, dst2d)


def _segsum_dual_sc(h2, srcab, dst2d):
    @functools.partial(
        pl.kernel,
        out_type=jax.ShapeDtypeStruct((_NC, _N, _D), jnp.float32),
        mesh=plsc.VectorSubcoreMesh(**_MESH),
        scratch_types=_SEG_SCRATCH,
    )
    def k(h_hbm, src_hbm, dst_hbm, out_hbm,
          sidx, didx, rows0, rows1, acc, sem0, sem1):
        c = lax.axis_index("c")
        s = lax.axis_index("s")
        _zero_acc(rows0, acc, s)
        plsc.subcore_barrier()

        def fetch(off, si, di):
            pltpu.sync_copy(src_hbm.at[c, pl.ds(off, _W)], si)
            pltpu.sync_copy(dst_hbm.at[pl.ds(off, _W)], di)

        _window_loop(h_hbm, fetch, s * _NT2, _NT2 // _W,
                     sidx, didx, rows0, rows1, sem0, sem1, acc)
        plsc.subcore_barrier()
        _acc_writeout(acc, out_hbm, c, s)

    return k(h2, srcab, dst2d)


def _gin_layer2_tc(h_and_spec, aggs, eps2, w1s, b1s, w2s, b2s,
                   premlp=None):
    n_agg = len(aggs[1])

    def body(*refs):
        eps_ref = refs[0]
        h_ref = refs[1]
        agg_refs = refs[2:2 + n_agg]
        w1_ref, b1_ref, w2_ref, b2_ref = refs[2 + n_agg:6 + n_agg]
        if premlp is None:
            (o_ref,) = refs[6 + n_agg:]
        else:
            pw_ref, pb_ref, o_ref, hp_ref, st_ref = refs[6 + n_agg:]
        agg = agg_refs[0][0]
        for r in agg_refs[1:]:
            agg = agg + r[0]
        z = (1.0 + eps_ref[0, 0]) * h_ref[...] + agg
        t = jnp.dot(z, w1_ref[0], preferred_element_type=jnp.float32)
        t = jnp.maximum(t + b1_ref[0], 0.0)
        o = jnp.dot(t, w2_ref[0], preferred_element_type=jnp.float32)
        o = jnp.maximum(o + b2_ref[0], 0.0)
        o_ref[...] = o

        if premlp is not None:
            j = pl.program_id(0)
            i = pl.program_id(1)

            @pl.when(jnp.logical_and(j == 0, i == 0))
            def _():
                st_ref[...] = jnp.zeros_like(st_ref)

            @pl.when(j == 0)
            def _():
                hp = jnp.dot(o, pw_ref[...],
                             preferred_element_type=jnp.float32) + pb_ref[...]
                hp_ref[...] = hp
                st_ref[0:1, :] += jnp.sum(hp, axis=0, keepdims=True)
                st_ref[1:2, :] += jnp.sum(hp * hp, axis=0, keepdims=True)

    h2, h_spec = h_and_spec
    agg_arr, agg_specs = aggs
    in_specs = [
        pl.BlockSpec(memory_space=pltpu.SMEM),
        h_spec,
        *agg_specs,
        pl.BlockSpec((1, _D, 2 * _D), lambda j, i: (j, 0, 0)),
        pl.BlockSpec((1, 1, 2 * _D), lambda j, i: (j, 0, 0)),
        pl.BlockSpec((1, 2 * _D, _D), lambda j, i: (j, 0, 0)),
        pl.BlockSpec((1, 1, _D), lambda j, i: (j, 0, 0)),
    ]
    out_specs = [pl.BlockSpec((_BN, _D), lambda j, i: (j * _NB + i, 0))]
    out_shape = [jax.ShapeDtypeStruct((2 * _N, _D), jnp.float32)]
    args = [eps2, h2, *agg_arr, w1s, b1s.reshape(2, 1, -1), w2s,
            b2s.reshape(2, 1, -1)]
    if premlp is not None:
        pw, pb = premlp
        in_specs += [
            pl.BlockSpec((_D, 2 * _D), lambda j, i: (0, 0)),
            pl.BlockSpec((1, 2 * _D), lambda j, i: (0, 0)),
        ]
        out_specs += [
            pl.BlockSpec((_BN, 2 * _D),
                         lambda j, i: (jnp.where(j == 0, i, _NB), 0)),
            pl.BlockSpec((8, 2 * _D), lambda j, i: (0, 0)),
        ]
        out_shape += [
            jax.ShapeDtypeStruct((_N + _BN, 2 * _D), jnp.float32),
            jax.ShapeDtypeStruct((8, 2 * _D), jnp.float32),
        ]
        args += [pw, pb.reshape(1, -1)]
    out = pl.pallas_call(
        body,
        grid=(2, _NB),
        in_specs=in_specs,
        out_specs=out_specs,
        out_shape=out_shape,
    )(*args)
    return out[0] if premlp is None else out


def _epilogue_tc(hp, stats, bn_g, bn_b, w2, b2, nf2, cb, batch2d,
                 cls_w, cls_b):
    def body(hp_ref, st_ref, g_ref, b_ref, w2_ref, b2_ref, nf_ref, cb_ref,
             bat_ref, w_ref, cb2_ref, cg_ref, sg_ref, sm_ref, cm_ref,
             logit_ref, cgo_ref, sgo_ref, cmt_ref, reg_ref):
        i = pl.program_id(0)

        @pl.when(i == 0)
        def _():
            cg_ref[...] = jnp.zeros_like(cg_ref)
            sg_ref[...] = jnp.zeros_like(sg_ref)
            sm_ref[...] = jnp.zeros_like(sm_ref)
            cm_ref[...] = jnp.zeros_like(cm_ref)

        m = st_ref[0:1, :] * (1.0 / _N)
        v = st_ref[1:2, :] * (1.0 / _N) - m * m
        hpn = (hp_ref[...] - m) * lax.rsqrt(v + 1e-5) * g_ref[...] + b_ref[...]
        hpn = jnp.maximum(hpn, 0.0)
        logits = jnp.dot(hpn, w2_ref[...],
                         preferred_element_type=jnp.float32) + b2_ref[...]
        score = jax.nn.sigmoid(logits)
        pos = jnp.mean(score, axis=1, keepdims=True)

        nfb = nf_ref[...]
        cbm = cb_ref[...]
        cbsq = lax.dot_general(jnp.ones((1, _D), jnp.float32), cbm * cbm,
                               (((1,), (1,)), ((), ())), precision=_HI,
                               preferred_element_type=jnp.float32)
        prox = lax.dot_general(nfb, cbm, (((1,), (1,)), ((), ())),
                               preferred_element_type=jnp.float32)
        dd = cbsq - 2.0 * prox
        minv = jnp.min(dd, axis=1, keepdims=True)
        ik = lax.broadcasted_iota(jnp.int32, dd.shape, 1)
        idx = jnp.min(jnp.where(dd == minv, ik, _K), axis=1, keepdims=True)
        onehot = (ik == idx).astype(jnp.float32)
        q = jnp.dot(onehot, cbm,
                    preferred_element_type=jnp.float32)

        diff = nfb - q
        res = nfb + q
        cpart = res * score
        spart = res * (1.0 - score)

        oh = (lax.broadcasted_iota(jnp.int32, (hp_ref.shape[0], _G), 1)
              == bat_ref[...]).astype(jnp.float32)
        cg_ref[...] += lax.dot_general(oh, cpart, (((0,), (0,)), ((), ())),
                                       preferred_element_type=jnp.float32)
        sg_ref[...] += lax.dot_general(oh, spart, (((0,), (0,)), ((), ())),
                                       preferred_element_type=jnp.float32)
        ph = pos.astype(jnp.bfloat16).astype(jnp.float32)
        rhs = jnp.concatenate(
            [ph, pos - ph, jnp.ones_like(pos),
             jnp.zeros((pos.shape[0], 5), jnp.float32)], axis=1)
        sm_ref[...] += lax.dot_general(oh, rhs, (((0,), (0,)), ((), ())),
                                       preferred_element_type=jnp.float32)
        cm_ref[...] += jnp.full((1, _D), jnp.sum(diff * diff))

        @pl.when(i == _NB - 1)
        def _():
            pos_sum = sm_ref[:, 0:1] + sm_ref[:, 1:2]
            pos_b = pos_sum + 1e-8
            neg_b = sm_ref[:, 2:3] - pos_sum + 1e-8
            inv = 1.0 / jnp.maximum(sm_ref[:, 2:3], 1.0)
            cg = cg_ref[...] * inv
            sg = sg_ref[...] * inv
            cgo_ref[...] = cg
            sgo_ref[...] = sg
            logit_ref[...] = jnp.dot(
                cg, w_ref[...], preferred_element_type=jnp.float32) + cb2_ref[...]
            reg = jnp.mean(jnp.abs(pos_b / (pos_b + neg_b) - _GAMMA))
            reg_ref[...] = jnp.full((1, 1), reg)
            cmt_ref[...] = cm_ref[0:1, 0:1] * (_CW / (_N * _D))

    _c0 = lambda i: (0, 0)
    return pl.pallas_call(
        body,
        grid=(_NB,),
        in_specs=[
            pl.BlockSpec((_BN, 2 * _D), lambda i: (i, 0)),
            pl.BlockSpec((8, 2 * _D), _c0),
            pl.BlockSpec((1, 2 * _D), _c0),
            pl.BlockSpec((1, 2 * _D), _c0),
            pl.BlockSpec((2 * _D, _D), _c0),
            pl.BlockSpec((1, _D), _c0),
            pl.BlockSpec((_BN, _D), lambda i: (_NB + i, 0)),
            pl.BlockSpec((_K, _D), _c0),
            pl.BlockSpec((_BN, 1), lambda i: (i, 0)),
            pl.BlockSpec((_D, _C), _c0),
            pl.BlockSpec((1, _C), _c0),
        ],
        out_specs=[
            pl.BlockSpec((_G, _D), _c0),
            pl.BlockSpec((_G, _D), _c0),
            pl.BlockSpec((_G, 8), _c0),
            pl.BlockSpec((1, _D), _c0),
            pl.BlockSpec((_G, _C), _c0),
            pl.BlockSpec((_G, _D), _c0),
            pl.BlockSpec((_G, _D), _c0),
            pl.BlockSpec((1, 1), _c0),
            pl.BlockSpec((1, 1), _c0),
        ],
        out_shape=[
            jax.ShapeDtypeStruct((_G, _D), jnp.float32),
            jax.ShapeDtypeStruct((_G, _D), jnp.float32),
            jax.ShapeDtypeStruct((_G, 8), jnp.float32),
            jax.ShapeDtypeStruct((1, _D), jnp.float32),
            jax.ShapeDtypeStruct((_G, _C), jnp.float32),
            jax.ShapeDtypeStruct((_G, _D), jnp.float32),
            jax.ShapeDtypeStruct((_G, _D), jnp.float32),
            jax.ShapeDtypeStruct((1, 1), jnp.float32),
            jax.ShapeDtypeStruct((1, 1), jnp.float32),
        ],
    )(hp, stats, bn_g.reshape(1, -1), bn_b.reshape(1, -1), w2,
      b2.reshape(1, -1), nf2, cb, batch2d, cls_w, cls_b.reshape(1, -1))


def _stack2(pa, pb, name):
    return jnp.stack([pa[name], pb[name]])


def kernel(x, edge_index, batch, params):
    npad = _EP - _E
    pad_src = jnp.arange(npad, dtype=jnp.int32) % _N
    pad_dst = _N + jnp.arange(npad, dtype=jnp.int32) % (_ACCR - _N)
    src2d = jnp.concatenate(
        [edge_index[0].astype(jnp.int32), pad_src]).reshape(_NCHP, _CHUNK)
    dst2d = jnp.concatenate(
        [edge_index[1].astype(jnp.int32), pad_dst]).reshape(_NCHP, _CHUNK)
    srcab = jnp.stack([src2d, src2d + _N])
    batch2d = batch.reshape(_N, 1).astype(jnp.int32)

    sep1, sep2 = params["sep_gnn"]
    enc1, enc2 = params["enc_gnn"]
    eps1 = jnp.stack([sep1["eps"], enc1["eps"]]).reshape(2, 1).astype(jnp.float32)
    eps2 = jnp.stack([sep2["eps"], enc2["eps"]]).reshape(2, 1).astype(jnp.float32)

    aggx = _segsum_sc(x, src2d, dst2d)
    h2 = _gin_layer2_tc(
        (x, pl.BlockSpec((_BN, _D), lambda j, i: (i, 0))),
        ([aggx, aggx],
         [pl.BlockSpec((1, _BN, _D), lambda j, i: (0, i, 0)),
          pl.BlockSpec((1, _BN, _D), lambda j, i: (1, i, 0))]),
        eps1, _stack2(sep1, enc1, "w1"), _stack2(sep1, enc1, "b1"),
        _stack2(sep1, enc1, "w2"), _stack2(sep1, enc1, "b2"))
    agg2 = _segsum_dual_sc(h2, srcab, dst2d)
    nf2, hp, stats = _gin_layer2_tc(
        (h2, pl.BlockSpec((_BN, _D), lambda j, i: (j * _NB + i, 0))),
        ([agg2], [pl.BlockSpec((1, _BN, _D), lambda j, i: (j, i, 0))]),
        eps2, _stack2(sep2, enc2, "w1"), _stack2(sep2, enc2, "b1"),
        _stack2(sep2, enc2, "w2"), _stack2(sep2, enc2, "b2"),
        premlp=(params["sep_w1"], params["sep_b1"]))

    outs = _epilogue_tc(
        hp, stats, params["sep_bn_g"], params["sep_bn_b"], params["sep_w2"],
        params["sep_b2"], nf2, params["codebook"], batch2d,
        params["cls_w"], params["cls_b"])
    logit, c_g, s_g, cmt, reg = outs[4:]
    return (logit, c_g, s_g, cmt[0, 0], reg[0, 0])
